# baseline (speedup 1.0000x reference)
import functools

import jax
import jax.numpy as jnp
from jax import lax
from jax.experimental import pallas as pl
from jax.experimental.pallas import tpu as pltpu
from jax.experimental.pallas import tpu_sc as plsc

N = 10000
E = 320000
D = 128
L = 3
C = 40

NPAD = 10240
ROWS_PER_TILE = NPAD // 16
NW = 32
CH = 128
EPW = NPAD
NCH = EPW // CH
TRASH = N

EPT = NPAD
PCH_CH = 64
PCH = EPT // PCH_CH
DEPTH = 4
NBLK = 4
BCH = PCH // NBLK

_SC_MESH = plsc.VectorSubcoreMesh(core_axis_name="c", subcore_axis_name="s")


def _sc_degree(dst3, ones_hbm, zeros16_hbm):
    @functools.partial(
        pl.kernel,
        out_type=jax.ShapeDtypeStruct((2, NPAD, 16), jnp.float32),
        mesh=_SC_MESH,
        scratch_types=[
            pltpu.VMEM((NCH, CH), jnp.int32),
            pltpu.VMEM((CH, 16), jnp.float32),
            pltpu.VMEM_SHARED((NPAD, 16), jnp.float32),
        ],
    )
    def deg_kernel(dst_hbm, ones_h, zeros_h, out_hbm, idx_v, ones_v, acc):
        cid = lax.axis_index("c")
        sid = lax.axis_index("s")
        wid = sid * 2 + cid
        base = sid * ROWS_PER_TILE
        pltpu.sync_copy(zeros_h, acc.at[pl.ds(base, ROWS_PER_TILE)])
        pltpu.sync_copy(ones_h, ones_v)
        pltpu.sync_copy(dst_hbm.at[wid], idx_v)
        plsc.subcore_barrier()

        def body(j, carry):
            pltpu.sync_copy(ones_v, acc.at[idx_v.at[j]], add=True)
            return carry

        lax.fori_loop(0, NCH, body, 0)
        plsc.subcore_barrier()
        pltpu.sync_copy(acc.at[pl.ds(base, ROWS_PER_TILE)],
                        out_hbm.at[cid, pl.ds(base, ROWS_PER_TILE)])

    return deg_kernel(dst3, ones_hbm, zeros16_hbm)


def _sc_propagate(m, src5, dst5, zeros_hbm):
    @functools.partial(
        pl.kernel,
        out_type=jax.ShapeDtypeStruct((2, NPAD, D), jnp.float32),
        mesh=_SC_MESH,
        scratch_types=[
            pltpu.VMEM((BCH, PCH_CH), jnp.int32),
            pltpu.VMEM((BCH, PCH_CH), jnp.int32),
            pltpu.VMEM((PCH_CH, D), jnp.float32),
            pltpu.VMEM((PCH_CH, D), jnp.float32),
            pltpu.VMEM((PCH_CH, D), jnp.float32),
            pltpu.VMEM((PCH_CH, D), jnp.float32),
            pltpu.VMEM_SHARED((NPAD, D), jnp.float32),
            pltpu.SemaphoreType.DMA,
            pltpu.SemaphoreType.DMA,
            pltpu.SemaphoreType.DMA,
            pltpu.SemaphoreType.DMA,
        ],
    )
    def prop_kernel(m_hbm, src_hbm, dst_hbm, zeros_h, out_hbm,
                    src_v, dst_v, rows0, rows1, rows2, rows3, acc,
                    sem0, sem1, sem2, sem3):
        cid = lax.axis_index("c")
        sid = lax.axis_index("s")
        base = sid * ROWS_PER_TILE
        pltpu.sync_copy(zeros_h, acc.at[pl.ds(base, ROWS_PER_TILE)])
        plsc.subcore_barrier()

        bufs = ((rows0, sem0), (rows1, sem1), (rows2, sem2), (rows3, sem3))

        def body(i, carry):
            j = DEPTH * i
            for k in range(DEPTH):
                r, s = bufs[k]
                pltpu.make_async_copy(m_hbm.at[src_v.at[j + k]], r, s).wait()
                pltpu.sync_copy(r, acc.at[dst_v.at[j + k]], add=True)
                pltpu.async_copy(m_hbm.at[src_v.at[j + k + DEPTH]], r, s)
            return carry

        for b in range(NBLK):
            pltpu.sync_copy(src_hbm.at[cid, sid, b], src_v)
            pltpu.sync_copy(dst_hbm.at[cid, sid, b], dst_v)
            for k in range(DEPTH):
                pltpu.async_copy(m_hbm.at[src_v.at[k]], bufs[k][0],
                                 bufs[k][1])
            lax.fori_loop(0, BCH // DEPTH - 1, body, 0)
            jlast = BCH - DEPTH
            for k in range(DEPTH):
                r, s = bufs[k]
                pltpu.make_async_copy(m_hbm.at[src_v.at[jlast + k]],
                                     r, s).wait()
                pltpu.sync_copy(r, acc.at[dst_v.at[jlast + k]], add=True)
        plsc.subcore_barrier()
        pltpu.sync_copy(acc.at[pl.ds(base, ROWS_PER_TILE)],
                        out_hbm.at[cid, pl.ds(base, ROWS_PER_TILE)])

    return prop_kernel(m, src5, dst5, zeros_hbm)


_RB = 1024
_GRID = NPAD // _RB

_row_spec = pl.BlockSpec((_RB, D), lambda i: (i, 0))
_full_spec = pl.BlockSpec((D, D), lambda i: (0, 0))
_vec_spec = pl.BlockSpec((1, D), lambda i: (0, 0))
_deg_spec = pl.BlockSpec((2, _RB, 16), lambda i: (0, i, 0))
_part_spec = pl.BlockSpec((2, _RB, D), lambda i: (0, i, 0))


def _ln(x, g, b):
    mu = jnp.mean(x, axis=-1, keepdims=True)
    var = jnp.mean((x - mu) ** 2, axis=-1, keepdims=True)
    return (x - mu) / jnp.sqrt(var + 1e-05) * g + b


def _silu(x):
    return x * jax.nn.sigmoid(x)


def _dinv_from(dp):
    deg = dp[0, :, 0:1] + dp[1, :, 0:1] + 1.0
    return lax.rsqrt(deg)


def _tc_h_kernel(x_ref, ws_ref, bs_ref, h_out):
    h = jnp.dot(x_ref[...], ws_ref[...], preferred_element_type=jnp.float32)
    h_out[...] = _silu(h + bs_ref[...])


def _tc_h(x_pad, W_start, b_start):
    return pl.pallas_call(
        _tc_h_kernel,
        grid=(_GRID,),
        in_specs=[_row_spec, _full_spec, _vec_spec],
        out_specs=_row_spec,
        out_shape=jax.ShapeDtypeStruct((NPAD, D), jnp.float32),
    )(x_pad, W_start, b_start)


def _tc_m_kernel(h_ref, dp_ref, g_ref, b_ref, w_ref, m_out):
    c = _ln(h_ref[...], g_ref[...], b_ref[...])
    m_out[...] = _dinv_from(dp_ref) * jnp.dot(
        c, w_ref[...], preferred_element_type=jnp.float32)


def _tc_m(h, degpart, g0, b0, gcn_W0):
    return pl.pallas_call(
        _tc_m_kernel,
        grid=(_GRID,),
        in_specs=[_row_spec, _deg_spec, _vec_spec, _vec_spec, _full_spec],
        out_specs=_row_spec,
        out_shape=jax.ShapeDtypeStruct((NPAD, D), jnp.float32),
    )(h, degpart, g0, b0, gcn_W0)


def _layer_update(h_ref, m_ref, part_ref, dp_ref, gb_ref, ag_ref,
                  ffg_ref, ffb_ref, ffw_ref, ffb2_ref, af_ref):
    dinv = _dinv_from(dp_ref)
    agg = (part_ref[0].astype(jnp.float32) + part_ref[1].astype(jnp.float32)
           + m_ref[...].astype(jnp.float32))
    c2 = _silu(dinv * agg + gb_ref[...])
    h = ag_ref[...] * c2 + h_ref[...]
    nx = _ln(h, ffg_ref[...], ffb_ref[...])
    f = _silu(jnp.dot(nx, ffw_ref[...], preferred_element_type=jnp.float32)
              + ffb2_ref[...])
    return af_ref[...] * f + h, dinv


def _tc_layer_kernel(h_ref, m_ref, part_ref, dp_ref, gb_ref, ag_ref,
                     ffg_ref, ffb_ref, ffw_ref, ffb2_ref, af_ref,
                     ng_ref, nb_ref, nw_ref, h_out, m_out):
    h, dinv = _layer_update(h_ref, m_ref, part_ref, dp_ref, gb_ref, ag_ref,
                            ffg_ref, ffb_ref, ffw_ref, ffb2_ref, af_ref)
    c = _ln(h, ng_ref[...], nb_ref[...])
    h_out[...] = h
    m_out[...] = (dinv * jnp.dot(c, nw_ref[...],
                                 preferred_element_type=jnp.float32)
                  ).astype(jnp.float32)


def _tc_layer(h, m, part, degpart, gb, ag, ffg, ffb, ffw, ffb2, af,
              ng, nb, nw):
    return pl.pallas_call(
        _tc_layer_kernel,
        grid=(_GRID,),
        in_specs=[_row_spec, _row_spec, _part_spec, _deg_spec, _vec_spec,
                  _vec_spec, _vec_spec, _vec_spec, _full_spec, _vec_spec,
                  _vec_spec, _vec_spec, _vec_spec, _full_spec],
        out_specs=[_row_spec, _row_spec],
        out_shape=[jax.ShapeDtypeStruct((NPAD, D), jnp.float32),
                   jax.ShapeDtypeStruct((NPAD, D), jnp.float32)],
    )(h, m, part, degpart, gb, ag, ffg, ffb, ffw, ffb2, af, ng, nb, nw)


def _tc_final_kernel(h_ref, m_ref, part_ref, dp_ref, gb_ref, ag_ref,
                     ffg_ref, ffb_ref, ffw_ref, ffb2_ref, af_ref,
                     wf_ref, bf_ref, out_ref):
    h, _ = _layer_update(h_ref, m_ref, part_ref, dp_ref, gb_ref, ag_ref,
                         ffg_ref, ffb_ref, ffw_ref, ffb2_ref, af_ref)
    logits = jnp.dot(h, wf_ref[...], preferred_element_type=jnp.float32)
    logits = logits + bf_ref[...]
    col = lax.broadcasted_iota(jnp.int32, logits.shape, 1)
    valid = col < C
    neg = jnp.float32(-1e30)
    mx = jnp.max(jnp.where(valid, logits, neg), axis=1, keepdims=True)
    s = jnp.sum(jnp.where(valid, jnp.exp(logits - mx), 0.0), axis=1,
                keepdims=True)
    out_ref[...] = logits - (mx + jnp.log(s))


def _tc_final(h, m, part, degpart, gb, ag, ffg, ffb, ffw, ffb2, af, wf, bf):
    return pl.pallas_call(
        _tc_final_kernel,
        grid=(_GRID,),
        in_specs=[_row_spec, _row_spec, _part_spec, _deg_spec, _vec_spec,
                  _vec_spec, _vec_spec, _vec_spec, _full_spec, _vec_spec,
                  _vec_spec, _full_spec, _vec_spec],
        out_specs=_row_spec,
        out_shape=jax.ShapeDtypeStruct((NPAD, D), jnp.float32),
    )(h, m, part, degpart, gb, ag, ffg, ffb, ffw, ffb2, af, wf, bf)


def kernel(x, edge_index, W_start, b_start, ln_gcn_g, ln_gcn_b, gcn_W, gcn_b,
           alpha_gcn, ln_ff_g, ln_ff_b, ffw_W, ffw_b, alpha_ff,
           W_final, b_final):
    f32 = jnp.float32
    src = edge_index[0].astype(jnp.int32)
    dst = edge_index[1].astype(jnp.int32)

    pad_w = EPW - (E // NW)
    dst3 = jnp.pad(dst.reshape(NW, E // NW), ((0, 0), (0, pad_w)),
                   constant_values=TRASH).reshape(NW, NCH, CH)

    pad_t = EPT - (E // 32)
    src5 = jnp.pad(src.reshape(32, E // 32), ((0, 0), (0, pad_t)),
                   constant_values=0).reshape(2, 16, NBLK, BCH, PCH_CH)
    dst5 = jnp.pad(dst.reshape(32, E // 32), ((0, 0), (0, pad_t)),
                   constant_values=TRASH).reshape(2, 16, NBLK, BCH, PCH_CH)

    x_pad = jnp.pad(x.astype(f32), ((0, NPAD - N), (0, 0)))
    zeros_hbm = jnp.zeros((ROWS_PER_TILE, D), jnp.float32)
    zeros16_hbm = jnp.zeros((ROWS_PER_TILE, 16), f32)
    ones_hbm = jnp.ones((CH, 16), f32)

    degpart = _sc_degree(dst3, ones_hbm, zeros16_hbm)

    row = lambda a: a.reshape(1, D)
    scal = lambda a: jnp.full((1, D), a, f32)

    h = _tc_h(x_pad, W_start.astype(f32), row(b_start))
    m = _tc_m(h, degpart, row(ln_gcn_g[0]), row(ln_gcn_b[0]), gcn_W[0])

    for i in range(L - 1):
        part = _sc_propagate(m, src5, dst5, zeros_hbm)
        h, m = _tc_layer(h, m, part, degpart,
                         row(gcn_b[i]), scal(alpha_gcn[i]),
                         row(ln_ff_g[i]), row(ln_ff_b[i]), ffw_W[i],
                         row(ffw_b[i]), scal(alpha_ff[i]),
                         row(ln_gcn_g[i + 1]), row(ln_gcn_b[i + 1]),
                         gcn_W[i + 1])

    part = _sc_propagate(m, src5, dst5, zeros_hbm)
    wf_pad = jnp.pad(W_final.astype(f32), ((0, 0), (0, D - C)))
    bf_pad = jnp.pad(b_final.astype(f32), (0, D - C)).reshape(1, D)
    out = _tc_final(h, m, part, degpart,
                    row(gcn_b[L - 1]), scal(alpha_gcn[L - 1]),
                    row(ln_ff_g[L - 1]), row(ln_ff_b[L - 1]), ffw_W[L - 1],
                    row(ffw_b[L - 1]), scal(alpha_ff[L - 1]),
                    wf_pad, bf_pad)
    return out[:N, :C]


# speedup vs baseline: 2.6587x; 2.6587x over previous
import functools

import jax
import jax.numpy as jnp
from jax import lax
from jax.experimental import pallas as pl
from jax.experimental.pallas import tpu as pltpu
from jax.experimental.pallas import tpu_sc as plsc

N = 10000
E = 320000
D = 128
L = 3
C = 40

NPAD = 10240
ROWS_PER_TILE = NPAD // 16
NW = 32
CH = 125
NCH = 80

PCH_CH = CH
PCH = NCH
DEPTH = 2
NBLK = 4
BCH = PCH // NBLK

_SC_MESH = plsc.VectorSubcoreMesh(core_axis_name="c", subcore_axis_name="s")


def _sc_degree(dst3, ones_hbm, zeros16_hbm):
    @functools.partial(
        pl.kernel,
        out_type=jax.ShapeDtypeStruct((2, NPAD, 16), jnp.float32),
        mesh=_SC_MESH,
        scratch_types=[
            pltpu.VMEM((NCH, CH), jnp.int32),
            pltpu.VMEM((CH, 16), jnp.float32),
            pltpu.VMEM_SHARED((NPAD, 16), jnp.float32),
        ],
    )
    def deg_kernel(dst_hbm, ones_h, zeros_h, out_hbm, idx_v, ones_v, acc):
        cid = lax.axis_index("c")
        sid = lax.axis_index("s")
        wid = sid * 2 + cid
        base = sid * ROWS_PER_TILE
        pltpu.sync_copy(zeros_h, acc.at[pl.ds(base, ROWS_PER_TILE)])
        pltpu.sync_copy(ones_h, ones_v)
        pltpu.sync_copy(dst_hbm.at[wid], idx_v)
        plsc.subcore_barrier()

        def body(j, carry):
            pltpu.sync_copy(ones_v, acc.at[idx_v.at[j]], add=True)
            return carry

        lax.fori_loop(0, NCH, body, 0)
        plsc.subcore_barrier()
        pltpu.sync_copy(acc.at[pl.ds(base, ROWS_PER_TILE)],
                        out_hbm.at[cid, pl.ds(base, ROWS_PER_TILE)])

    return deg_kernel(dst3, ones_hbm, zeros16_hbm)


def _sc_propagate(m, src5, dst5, zeros_hbm):
    @functools.partial(
        pl.kernel,
        out_type=jax.ShapeDtypeStruct((2, NPAD, D), jnp.float32),
        mesh=_SC_MESH,
        scratch_types=[
            pltpu.VMEM((BCH, PCH_CH), jnp.int32),
            pltpu.VMEM((BCH, PCH_CH), jnp.int32),
            pltpu.VMEM((PCH_CH, D), jnp.float32),
            pltpu.VMEM((PCH_CH, D), jnp.float32),
            pltpu.VMEM_SHARED((NPAD, D), jnp.float32),
            pltpu.SemaphoreType.DMA,
            pltpu.SemaphoreType.DMA,
        ],
    )
    def prop_kernel(m_hbm, src_hbm, dst_hbm, zeros_h, out_hbm,
                    src_v, dst_v, rows0, rows1, acc, sem0, sem1):
        cid = lax.axis_index("c")
        sid = lax.axis_index("s")
        base = sid * ROWS_PER_TILE
        pltpu.sync_copy(zeros_h, acc.at[pl.ds(base, ROWS_PER_TILE)])
        plsc.subcore_barrier()

        bufs = ((rows0, sem0), (rows1, sem1))

        def body(i, carry):
            j = DEPTH * i
            for k in range(DEPTH):
                r, s = bufs[k]
                pltpu.make_async_copy(m_hbm.at[src_v.at[j + k]], r, s).wait()
                pltpu.sync_copy(r, acc.at[dst_v.at[j + k]], add=True)
                pltpu.async_copy(m_hbm.at[src_v.at[j + k + DEPTH]], r, s)
            return carry

        for b in range(NBLK):
            pltpu.sync_copy(src_hbm.at[cid, sid, b], src_v)
            pltpu.sync_copy(dst_hbm.at[cid, sid, b], dst_v)
            for k in range(DEPTH):
                pltpu.async_copy(m_hbm.at[src_v.at[k]], bufs[k][0],
                                 bufs[k][1])
            lax.fori_loop(0, BCH // DEPTH - 1, body, 0)
            jlast = BCH - DEPTH
            for k in range(DEPTH):
                r, s = bufs[k]
                pltpu.make_async_copy(m_hbm.at[src_v.at[jlast + k]],
                                     r, s).wait()
                pltpu.sync_copy(r, acc.at[dst_v.at[jlast + k]], add=True)
        plsc.subcore_barrier()
        pltpu.sync_copy(acc.at[pl.ds(base, ROWS_PER_TILE)],
                        out_hbm.at[cid, pl.ds(base, ROWS_PER_TILE)])

    return prop_kernel(m, src5, dst5, zeros_hbm)


_RB = 1024
_GRID = NPAD // _RB

_row_spec = pl.BlockSpec((_RB, D), lambda i: (i, 0))
_full_spec = pl.BlockSpec((D, D), lambda i: (0, 0))
_vec_spec = pl.BlockSpec((1, D), lambda i: (0, 0))
_deg_spec = pl.BlockSpec((2, _RB, 16), lambda i: (0, i, 0))
_part_spec = pl.BlockSpec((2, _RB, D), lambda i: (0, i, 0))


def _ln(x, g, b):
    mu = jnp.mean(x, axis=-1, keepdims=True)
    var = jnp.mean((x - mu) ** 2, axis=-1, keepdims=True)
    return (x - mu) / jnp.sqrt(var + 1e-05) * g + b


def _silu(x):
    return x * jax.nn.sigmoid(x)


def _dinv_from(dp):
    deg = dp[0, :, 0:1] + dp[1, :, 0:1] + 1.0
    return lax.rsqrt(deg)


def _tc_h_kernel(x_ref, ws_ref, bs_ref, h_out):
    h = jnp.dot(x_ref[...], ws_ref[...], preferred_element_type=jnp.float32)
    h_out[...] = _silu(h + bs_ref[...])


def _tc_h(x_pad, W_start, b_start):
    return pl.pallas_call(
        _tc_h_kernel,
        grid=(_GRID,),
        in_specs=[_row_spec, _full_spec, _vec_spec],
        out_specs=_row_spec,
        out_shape=jax.ShapeDtypeStruct((NPAD, D), jnp.float32),
    )(x_pad, W_start, b_start)


def _tc_m_kernel(h_ref, dp_ref, g_ref, b_ref, w_ref, m_out):
    c = _ln(h_ref[...], g_ref[...], b_ref[...])
    m_out[...] = _dinv_from(dp_ref) * jnp.dot(
        c, w_ref[...], preferred_element_type=jnp.float32)


def _tc_m(h, degpart, g0, b0, gcn_W0):
    return pl.pallas_call(
        _tc_m_kernel,
        grid=(_GRID,),
        in_specs=[_row_spec, _deg_spec, _vec_spec, _vec_spec, _full_spec],
        out_specs=_row_spec,
        out_shape=jax.ShapeDtypeStruct((NPAD, D), jnp.float32),
    )(h, degpart, g0, b0, gcn_W0)


def _layer_update(h_ref, m_ref, part_ref, dp_ref, gb_ref, ag_ref,
                  ffg_ref, ffb_ref, ffw_ref, ffb2_ref, af_ref):
    dinv = _dinv_from(dp_ref)
    agg = (part_ref[0].astype(jnp.float32) + part_ref[1].astype(jnp.float32)
           + m_ref[...].astype(jnp.float32))
    c2 = _silu(dinv * agg + gb_ref[...])
    h = ag_ref[...] * c2 + h_ref[...]
    nx = _ln(h, ffg_ref[...], ffb_ref[...])
    f = _silu(jnp.dot(nx, ffw_ref[...], preferred_element_type=jnp.float32)
              + ffb2_ref[...])
    return af_ref[...] * f + h, dinv


def _tc_layer_kernel(h_ref, m_ref, part_ref, dp_ref, gb_ref, ag_ref,
                     ffg_ref, ffb_ref, ffw_ref, ffb2_ref, af_ref,
                     ng_ref, nb_ref, nw_ref, h_out, m_out):
    h, dinv = _layer_update(h_ref, m_ref, part_ref, dp_ref, gb_ref, ag_ref,
                            ffg_ref, ffb_ref, ffw_ref, ffb2_ref, af_ref)
    c = _ln(h, ng_ref[...], nb_ref[...])
    h_out[...] = h
    m_out[...] = (dinv * jnp.dot(c, nw_ref[...],
                                 preferred_element_type=jnp.float32)
                  ).astype(jnp.float32)


def _tc_layer(h, m, part, degpart, gb, ag, ffg, ffb, ffw, ffb2, af,
              ng, nb, nw):
    return pl.pallas_call(
        _tc_layer_kernel,
        grid=(_GRID,),
        in_specs=[_row_spec, _row_spec, _part_spec, _deg_spec, _vec_spec,
                  _vec_spec, _vec_spec, _vec_spec, _full_spec, _vec_spec,
                  _vec_spec, _vec_spec, _vec_spec, _full_spec],
        out_specs=[_row_spec, _row_spec],
        out_shape=[jax.ShapeDtypeStruct((NPAD, D), jnp.float32),
                   jax.ShapeDtypeStruct((NPAD, D), jnp.float32)],
    )(h, m, part, degpart, gb, ag, ffg, ffb, ffw, ffb2, af, ng, nb, nw)


def _tc_final_kernel(h_ref, m_ref, part_ref, dp_ref, gb_ref, ag_ref,
                     ffg_ref, ffb_ref, ffw_ref, ffb2_ref, af_ref,
                     wf_ref, bf_ref, out_ref):
    h, _ = _layer_update(h_ref, m_ref, part_ref, dp_ref, gb_ref, ag_ref,
                         ffg_ref, ffb_ref, ffw_ref, ffb2_ref, af_ref)
    logits = jnp.dot(h, wf_ref[...], preferred_element_type=jnp.float32)
    logits = logits + bf_ref[...]
    col = lax.broadcasted_iota(jnp.int32, logits.shape, 1)
    valid = col < C
    neg = jnp.float32(-1e30)
    mx = jnp.max(jnp.where(valid, logits, neg), axis=1, keepdims=True)
    s = jnp.sum(jnp.where(valid, jnp.exp(logits - mx), 0.0), axis=1,
                keepdims=True)
    out_ref[...] = logits - (mx + jnp.log(s))


def _tc_final(h, m, part, degpart, gb, ag, ffg, ffb, ffw, ffb2, af, wf, bf):
    return pl.pallas_call(
        _tc_final_kernel,
        grid=(_GRID,),
        in_specs=[_row_spec, _row_spec, _part_spec, _deg_spec, _vec_spec,
                  _vec_spec, _vec_spec, _vec_spec, _full_spec, _vec_spec,
                  _vec_spec, _full_spec, _vec_spec],
        out_specs=_row_spec,
        out_shape=jax.ShapeDtypeStruct((NPAD, D), jnp.float32),
    )(h, m, part, degpart, gb, ag, ffg, ffb, ffw, ffb2, af, wf, bf)


def kernel(x, edge_index, W_start, b_start, ln_gcn_g, ln_gcn_b, gcn_W, gcn_b,
           alpha_gcn, ln_ff_g, ln_ff_b, ffw_W, ffw_b, alpha_ff,
           W_final, b_final):
    f32 = jnp.float32
    src = edge_index[0].astype(jnp.int32)
    dst = edge_index[1].astype(jnp.int32)

    dst3 = dst.reshape(NW, NCH, CH)
    src5 = src.reshape(2, 16, NBLK, BCH, PCH_CH)
    dst5 = dst.reshape(2, 16, NBLK, BCH, PCH_CH)

    x_pad = jnp.pad(x.astype(f32), ((0, NPAD - N), (0, 0)))
    zeros_hbm = jnp.zeros((ROWS_PER_TILE, D), jnp.float32)
    zeros16_hbm = jnp.zeros((ROWS_PER_TILE, 16), f32)
    ones_hbm = jnp.ones((CH, 16), f32)

    degpart = _sc_degree(dst3, ones_hbm, zeros16_hbm)

    row = lambda a: a.reshape(1, D)
    scal = lambda a: jnp.full((1, D), a, f32)

    h = _tc_h(x_pad, W_start.astype(f32), row(b_start))
    m = _tc_m(h, degpart, row(ln_gcn_g[0]), row(ln_gcn_b[0]), gcn_W[0])

    for i in range(L - 1):
        part = _sc_propagate(m, src5, dst5, zeros_hbm)
        h, m = _tc_layer(h, m, part, degpart,
                         row(gcn_b[i]), scal(alpha_gcn[i]),
                         row(ln_ff_g[i]), row(ln_ff_b[i]), ffw_W[i],
                         row(ffw_b[i]), scal(alpha_ff[i]),
                         row(ln_gcn_g[i + 1]), row(ln_gcn_b[i + 1]),
                         gcn_W[i + 1])

    part = _sc_propagate(m, src5, dst5, zeros_hbm)
    wf_pad = jnp.pad(W_final.astype(f32), ((0, 0), (0, D - C)))
    bf_pad = jnp.pad(b_final.astype(f32), (0, D - C)).reshape(1, D)
    out = _tc_final(h, m, part, degpart,
                    row(gcn_b[L - 1]), scal(alpha_gcn[L - 1]),
                    row(ln_ff_g[L - 1]), row(ln_ff_b[L - 1]), ffw_W[L - 1],
                    row(ffw_b[L - 1]), scal(alpha_ff[L - 1]),
                    wf_pad, bf_pad)
    return out[:N, :C]


# speedup vs baseline: 2.6746x; 1.0060x over previous
import functools

import jax
import jax.numpy as jnp
from jax import lax
from jax.experimental import pallas as pl
from jax.experimental.pallas import tpu as pltpu
from jax.experimental.pallas import tpu_sc as plsc

N = 10000
E = 320000
D = 128
L = 3
C = 40

NPAD = 10240
ROWS_PER_TILE = NPAD // 16
NW = 32
CH = 125
NCH = 80

PCH_CH = CH
PCH = NCH
DEPTH = 2
NBLK = 4
BCH = PCH // NBLK

_SC_MESH = plsc.VectorSubcoreMesh(core_axis_name="c", subcore_axis_name="s")


def _sc_degree(dst3, ones_hbm, zeros16_hbm):
    @functools.partial(
        pl.kernel,
        out_type=jax.ShapeDtypeStruct((2, NPAD, 16), jnp.float32),
        mesh=_SC_MESH,
        scratch_types=[
            pltpu.VMEM((NCH, CH), jnp.int32),
            pltpu.VMEM((CH, 16), jnp.float32),
            pltpu.VMEM_SHARED((NPAD, 16), jnp.float32),
        ],
    )
    def deg_kernel(dst_hbm, ones_h, zeros_h, out_hbm, idx_v, ones_v, acc):
        cid = lax.axis_index("c")
        sid = lax.axis_index("s")
        wid = sid * 2 + cid
        base = sid * ROWS_PER_TILE
        pltpu.sync_copy(zeros_h, acc.at[pl.ds(base, ROWS_PER_TILE)])
        pltpu.sync_copy(ones_h, ones_v)
        pltpu.sync_copy(dst_hbm.at[wid], idx_v)
        plsc.subcore_barrier()

        def body(j, carry):
            pltpu.sync_copy(ones_v, acc.at[idx_v.at[j]], add=True)
            return carry

        lax.fori_loop(0, NCH, body, 0)
        plsc.subcore_barrier()
        pltpu.sync_copy(acc.at[pl.ds(base, ROWS_PER_TILE)],
                        out_hbm.at[cid, pl.ds(base, ROWS_PER_TILE)])

    return deg_kernel(dst3, ones_hbm, zeros16_hbm)


def _sc_propagate(m, src5, dst5, zeros_hbm):
    @functools.partial(
        pl.kernel,
        out_type=jax.ShapeDtypeStruct((2, NPAD, D), jnp.float32),
        mesh=_SC_MESH,
        scratch_types=[
            pltpu.VMEM((BCH, PCH_CH), jnp.int32),
            pltpu.VMEM((BCH, PCH_CH), jnp.int32),
            pltpu.VMEM((PCH_CH, D), jnp.float32),
            pltpu.VMEM((PCH_CH, D), jnp.float32),
            pltpu.VMEM_SHARED((NPAD, D), jnp.float32),
            pltpu.SemaphoreType.DMA,
            pltpu.SemaphoreType.DMA,
        ],
    )
    def prop_kernel(m_hbm, src_hbm, dst_hbm, zeros_h, out_hbm,
                    src_v, dst_v, rows0, rows1, acc, sem0, sem1):
        cid = lax.axis_index("c")
        sid = lax.axis_index("s")
        base = sid * ROWS_PER_TILE
        pltpu.sync_copy(zeros_h, acc.at[pl.ds(base, ROWS_PER_TILE)])
        plsc.subcore_barrier()

        bufs = ((rows0, sem0), (rows1, sem1))

        def body(i, carry):
            j = DEPTH * i
            for k in range(DEPTH):
                r, s = bufs[k]
                pltpu.make_async_copy(m_hbm.at[src_v.at[j + k]], r, s).wait()
                pltpu.sync_copy(r, acc.at[dst_v.at[j + k]], add=True)
                pltpu.async_copy(m_hbm.at[src_v.at[j + k + DEPTH]], r, s)
            return carry

        for b in range(NBLK):
            pltpu.sync_copy(src_hbm.at[cid, sid, b], src_v)
            pltpu.sync_copy(dst_hbm.at[cid, sid, b], dst_v)
            for k in range(DEPTH):
                pltpu.async_copy(m_hbm.at[src_v.at[k]], bufs[k][0],
                                 bufs[k][1])
            lax.fori_loop(0, BCH // DEPTH - 1, body, 0)
            jlast = BCH - DEPTH
            for k in range(DEPTH):
                r, s = bufs[k]
                pltpu.make_async_copy(m_hbm.at[src_v.at[jlast + k]],
                                     r, s).wait()
                pltpu.sync_copy(r, acc.at[dst_v.at[jlast + k]], add=True)
        plsc.subcore_barrier()
        pltpu.sync_copy(acc.at[pl.ds(base, ROWS_PER_TILE)],
                        out_hbm.at[cid, pl.ds(base, ROWS_PER_TILE)])

    return prop_kernel(m, src5, dst5, zeros_hbm)


_RB = 2048
_GRID = NPAD // _RB

_row_spec = pl.BlockSpec((_RB, D), lambda i: (i, 0))
_full_spec = pl.BlockSpec((D, D), lambda i: (0, 0))
_vec_spec = pl.BlockSpec((1, D), lambda i: (0, 0))
_deg_spec = pl.BlockSpec((2, _RB, 16), lambda i: (0, i, 0))
_part_spec = pl.BlockSpec((2, _RB, D), lambda i: (0, i, 0))


def _ln(x, g, b):
    mu = jnp.mean(x, axis=-1, keepdims=True)
    var = jnp.mean((x - mu) ** 2, axis=-1, keepdims=True)
    return (x - mu) / jnp.sqrt(var + 1e-05) * g + b


def _silu(x):
    return x * jax.nn.sigmoid(x)


def _dinv_from(dp):
    deg = dp[0, :, 0:1] + dp[1, :, 0:1] + 1.0
    return lax.rsqrt(deg)


def _tc_start_kernel(x_ref, dp_ref, ws_ref, bs_ref, g_ref, b_ref, w_ref,
                     h_out, m_out):
    h = jnp.dot(x_ref[...], ws_ref[...], preferred_element_type=jnp.float32)
    h = _silu(h + bs_ref[...])
    c = _ln(h, g_ref[...], b_ref[...])
    m = _dinv_from(dp_ref) * jnp.dot(c, w_ref[...],
                                     preferred_element_type=jnp.float32)
    h_out[...] = h
    m_out[...] = m


def _tc_start(x_pad, degpart, W_start, b_start, g0, b0, gcn_W0):
    return pl.pallas_call(
        _tc_start_kernel,
        grid=(_GRID,),
        in_specs=[_row_spec, _deg_spec, _full_spec, _vec_spec, _vec_spec,
                  _vec_spec, _full_spec],
        out_specs=[_row_spec, _row_spec],
        out_shape=[jax.ShapeDtypeStruct((NPAD, D), jnp.float32),
                   jax.ShapeDtypeStruct((NPAD, D), jnp.float32)],
    )(x_pad, degpart, W_start, b_start, g0, b0, gcn_W0)


def _layer_update(h_ref, m_ref, part_ref, dp_ref, gb_ref, ag_ref,
                  ffg_ref, ffb_ref, ffw_ref, ffb2_ref, af_ref):
    dinv = _dinv_from(dp_ref)
    agg = (part_ref[0].astype(jnp.float32) + part_ref[1].astype(jnp.float32)
           + m_ref[...].astype(jnp.float32))
    c2 = _silu(dinv * agg + gb_ref[...])
    h = ag_ref[...] * c2 + h_ref[...]
    nx = _ln(h, ffg_ref[...], ffb_ref[...])
    f = _silu(jnp.dot(nx, ffw_ref[...], preferred_element_type=jnp.float32)
              + ffb2_ref[...])
    return af_ref[...] * f + h, dinv


def _tc_layer_kernel(h_ref, m_ref, part_ref, dp_ref, gb_ref, ag_ref,
                     ffg_ref, ffb_ref, ffw_ref, ffb2_ref, af_ref,
                     ng_ref, nb_ref, nw_ref, h_out, m_out):
    h, dinv = _layer_update(h_ref, m_ref, part_ref, dp_ref, gb_ref, ag_ref,
                            ffg_ref, ffb_ref, ffw_ref, ffb2_ref, af_ref)
    c = _ln(h, ng_ref[...], nb_ref[...])
    h_out[...] = h
    m_out[...] = (dinv * jnp.dot(c, nw_ref[...],
                                 preferred_element_type=jnp.float32)
                  ).astype(jnp.float32)


def _tc_layer(h, m, part, degpart, gb, ag, ffg, ffb, ffw, ffb2, af,
              ng, nb, nw):
    return pl.pallas_call(
        _tc_layer_kernel,
        grid=(_GRID,),
        in_specs=[_row_spec, _row_spec, _part_spec, _deg_spec, _vec_spec,
                  _vec_spec, _vec_spec, _vec_spec, _full_spec, _vec_spec,
                  _vec_spec, _vec_spec, _vec_spec, _full_spec],
        out_specs=[_row_spec, _row_spec],
        out_shape=[jax.ShapeDtypeStruct((NPAD, D), jnp.float32),
                   jax.ShapeDtypeStruct((NPAD, D), jnp.float32)],
    )(h, m, part, degpart, gb, ag, ffg, ffb, ffw, ffb2, af, ng, nb, nw)


def _tc_final_kernel(h_ref, m_ref, part_ref, dp_ref, gb_ref, ag_ref,
                     ffg_ref, ffb_ref, ffw_ref, ffb2_ref, af_ref,
                     wf_ref, bf_ref, out_ref):
    h, _ = _layer_update(h_ref, m_ref, part_ref, dp_ref, gb_ref, ag_ref,
                         ffg_ref, ffb_ref, ffw_ref, ffb2_ref, af_ref)
    logits = jnp.dot(h, wf_ref[...], preferred_element_type=jnp.float32)
    logits = logits + bf_ref[...]
    col = lax.broadcasted_iota(jnp.int32, logits.shape, 1)
    valid = col < C
    neg = jnp.float32(-1e30)
    mx = jnp.max(jnp.where(valid, logits, neg), axis=1, keepdims=True)
    s = jnp.sum(jnp.where(valid, jnp.exp(logits - mx), 0.0), axis=1,
                keepdims=True)
    out_ref[...] = logits - (mx + jnp.log(s))


def _tc_final(h, m, part, degpart, gb, ag, ffg, ffb, ffw, ffb2, af, wf, bf):
    return pl.pallas_call(
        _tc_final_kernel,
        grid=(_GRID,),
        in_specs=[_row_spec, _row_spec, _part_spec, _deg_spec, _vec_spec,
                  _vec_spec, _vec_spec, _vec_spec, _full_spec, _vec_spec,
                  _vec_spec, _full_spec, _vec_spec],
        out_specs=_row_spec,
        out_shape=jax.ShapeDtypeStruct((NPAD, D), jnp.float32),
    )(h, m, part, degpart, gb, ag, ffg, ffb, ffw, ffb2, af, wf, bf)


def kernel(x, edge_index, W_start, b_start, ln_gcn_g, ln_gcn_b, gcn_W, gcn_b,
           alpha_gcn, ln_ff_g, ln_ff_b, ffw_W, ffw_b, alpha_ff,
           W_final, b_final):
    f32 = jnp.float32
    src = edge_index[0].astype(jnp.int32)
    dst = edge_index[1].astype(jnp.int32)

    dst3 = dst.reshape(NW, NCH, CH)
    src5 = src.reshape(2, 16, NBLK, BCH, PCH_CH)
    dst5 = dst.reshape(2, 16, NBLK, BCH, PCH_CH)

    x_pad = jnp.pad(x.astype(f32), ((0, NPAD - N), (0, 0)))
    zeros_hbm = jnp.zeros((ROWS_PER_TILE, D), jnp.float32)
    zeros16_hbm = jnp.zeros((ROWS_PER_TILE, 16), f32)
    ones_hbm = jnp.ones((CH, 16), f32)

    degpart = _sc_degree(dst3, ones_hbm, zeros16_hbm)

    row = lambda a: a.reshape(1, D)
    scal = lambda a: jnp.full((1, D), a, f32)

    h, m = _tc_start(x_pad, degpart, W_start.astype(f32), row(b_start),
                     row(ln_gcn_g[0]), row(ln_gcn_b[0]), gcn_W[0])

    for i in range(L - 1):
        part = _sc_propagate(m, src5, dst5, zeros_hbm)
        h, m = _tc_layer(h, m, part, degpart,
                         row(gcn_b[i]), scal(alpha_gcn[i]),
                         row(ln_ff_g[i]), row(ln_ff_b[i]), ffw_W[i],
                         row(ffw_b[i]), scal(alpha_ff[i]),
                         row(ln_gcn_g[i + 1]), row(ln_gcn_b[i + 1]),
                         gcn_W[i + 1])

    part = _sc_propagate(m, src5, dst5, zeros_hbm)
    wf_pad = jnp.pad(W_final.astype(f32), ((0, 0), (0, D - C)))
    bf_pad = jnp.pad(b_final.astype(f32), (0, D - C)).reshape(1, D)
    out = _tc_final(h, m, part, degpart,
                    row(gcn_b[L - 1]), scal(alpha_gcn[L - 1]),
                    row(ln_ff_g[L - 1]), row(ln_ff_b[L - 1]), ffw_W[L - 1],
                    row(ffw_b[L - 1]), scal(alpha_ff[L - 1]),
                    wf_pad, bf_pad)
    return out[:N, :C]
